# baseline (speedup 1.0000x reference)
import jax
import jax.numpy as jnp
from jax.experimental import pallas as pl
from jax.experimental.pallas import tpu as pltpu

_N = 320000
_D = 128
_BLK = 4000


def _sum_body(a_ref, b_ref, out_ref):
    step = pl.program_id(0)
    partial = jnp.sum(a_ref[...], axis=0, keepdims=True) + jnp.sum(
        b_ref[...], axis=0, keepdims=True
    )

    @pl.when(step == 0)
    def _init():
        out_ref[...] = partial

    @pl.when(step != 0)
    def _acc():
        out_ref[...] += partial


def kernel(attrs_0, attrs_1):
    grid = _N // _BLK
    out = pl.pallas_call(
        _sum_body,
        grid=(grid,),
        in_specs=[
            pl.BlockSpec((_BLK, _D), lambda i: (i, 0)),
            pl.BlockSpec((_BLK, _D), lambda i: (i, 0)),
        ],
        out_specs=pl.BlockSpec((1, _D), lambda i: (0, 0)),
        out_shape=jax.ShapeDtypeStruct((1, _D), jnp.float32),
    )(attrs_0, attrs_1)
    return out.reshape(_D)


# speedup vs baseline: 1.1960x; 1.1960x over previous
import jax
import jax.numpy as jnp
from jax.experimental import pallas as pl
from jax.experimental.pallas import tpu as pltpu

_N = 320000
_D = 128
_BLK = 8000


def _sum_body(a_ref, b_ref, out_ref):
    step = pl.program_id(0)
    a = a_ref[...].reshape(_BLK // 8, 8, _D)
    b = b_ref[...].reshape(_BLK // 8, 8, _D)
    partial = jnp.sum(a, axis=0) + jnp.sum(b, axis=0)

    @pl.when(step == 0)
    def _init():
        out_ref[...] = partial

    @pl.when(step != 0)
    def _acc():
        out_ref[...] += partial


def kernel(attrs_0, attrs_1):
    grid = _N // _BLK
    out = pl.pallas_call(
        _sum_body,
        grid=(grid,),
        in_specs=[
            pl.BlockSpec((_BLK, _D), lambda i: (i, 0)),
            pl.BlockSpec((_BLK, _D), lambda i: (i, 0)),
        ],
        out_specs=pl.BlockSpec((8, _D), lambda i: (0, 0)),
        out_shape=jax.ShapeDtypeStruct((8, _D), jnp.float32),
    )(attrs_0, attrs_1)
    return jnp.sum(out, axis=0)
